# baseline (speedup 1.0000x reference)
import functools

import jax
import jax.numpy as jnp
from jax import lax
from jax.experimental import pallas as pl
from jax.experimental.pallas import tpu as pltpu
from jax.experimental.pallas import tpu_sc as plsc

_INFO = plsc.get_sparse_core_info()
_NC, _NS = _INFO.num_cores, _INFO.num_subcores
_NW = _NC * _NS

_B = 4096 * 200
_D = 64
_DP = 128
_BPW = _B // _NW
_C = 200
_STEPS = _BPW // _C
_NBUF = 4


@functools.partial(
    pl.kernel,
    out_type=jax.ShapeDtypeStruct((_B, _DP), jnp.float32),
    mesh=plsc.VectorSubcoreMesh(core_axis_name="c", subcore_axis_name="s"),
    scratch_types=[
        pltpu.VMEM((_BPW,), jnp.int32),
        pltpu.VMEM((_NBUF, _C, _DP), jnp.float32),
    ]
    + [pltpu.SemaphoreType.DMA] * (2 * _NBUF),
)
def _gather_kernel(table_hbm, idx_hbm, out_hbm, idx_v, rows_v, *sems):
    gsem = sems[:_NBUF]
    wsem = sems[_NBUF:]
    wid = lax.axis_index("s") * _NC + lax.axis_index("c")
    base = wid * _BPW

    pltpu.sync_copy(idx_hbm.at[pl.ds(base, _BPW)], idx_v)

    def start_gather(m, b):
        pltpu.async_copy(
            table_hbm.at[idx_v.at[pl.ds(m * _C, _C)]], rows_v.at[b], gsem[b]
        )

    def wait_gather(m, b):
        pltpu.make_async_copy(
            table_hbm.at[idx_v.at[pl.ds(m * _C, _C)]], rows_v.at[b], gsem[b]
        ).wait()

    def start_wb(m, b):
        pltpu.async_copy(
            rows_v.at[b], out_hbm.at[pl.ds(base + m * _C, _C)], wsem[b]
        )

    def wait_wb(m, b):
        pltpu.make_async_copy(
            rows_v.at[b], out_hbm.at[pl.ds(base + m * _C, _C)], wsem[b]
        ).wait()

    for j in range(_NBUF - 1):
        start_gather(j, j)

    def outer(g, carry):
        for b_s in range(_NBUF):
            m = g * _NBUF + b_s
            bn = (b_s + _NBUF - 1) % _NBUF

            @pl.when(m + (_NBUF - 1) < _STEPS)
            def _():
                @pl.when(m >= 1)
                def _():
                    wait_wb(m - 1, bn)

                start_gather(m + (_NBUF - 1), bn)

            wait_gather(m, b_s)
            start_wb(m, b_s)
        return carry

    lax.fori_loop(0, _STEPS // _NBUF, outer, 0)

    for j in range(_NBUF):
        wait_wb(_STEPS - _NBUF + j, (_STEPS - _NBUF + j) % _NBUF)


def kernel(input_ids, table):
    ids_flat = input_ids.reshape(-1).astype(jnp.int32)
    t128 = table.reshape(table.shape[0] // 2, _DP)
    pair = _gather_kernel(t128, ids_flat >> 1)
    odd = (ids_flat & 1).astype(jnp.bool_)[:, None]
    out = jnp.where(odd, pair[:, _D:], pair[:, :_D])
    return out.reshape(input_ids.shape + (table.shape[1],))


# speedup vs baseline: 1.3806x; 1.3806x over previous
import functools

import jax
import jax.numpy as jnp
from jax import lax
from jax.experimental import pallas as pl
from jax.experimental.pallas import tpu as pltpu
from jax.experimental.pallas import tpu_sc as plsc

_INFO = plsc.get_sparse_core_info()
_NC, _NS = _INFO.num_cores, _INFO.num_subcores
_NW = _NC * _NS

_B = 4096 * 200
_D = 64
_DP = 128
_BPW = _B // _NW
_C = 200
_STEPS = _BPW // _C
_NBUF = 4


@functools.partial(
    pl.kernel,
    out_type=jax.ShapeDtypeStruct((_B, _DP), jnp.float32),
    mesh=plsc.VectorSubcoreMesh(core_axis_name="c", subcore_axis_name="s"),
    scratch_types=[
        pltpu.VMEM((_BPW,), jnp.int32),
        pltpu.VMEM((_NBUF, _C, _DP), jnp.float32),
    ]
    + [pltpu.SemaphoreType.DMA] * (2 * _NBUF),
)
def _gather_kernel(table_hbm, idx_hbm, out_hbm, idx_v, rows_v, *sems):
    gsem = sems[:_NBUF]
    wsem = sems[_NBUF:]
    wid = lax.axis_index("s") * _NC + lax.axis_index("c")
    base = wid * _BPW

    pltpu.sync_copy(idx_hbm.at[pl.ds(base, _BPW)], idx_v)

    def start_gather(m, b):
        pltpu.async_copy(
            table_hbm.at[idx_v.at[pl.ds(m * _C, _C)]], rows_v.at[b], gsem[b]
        )

    def wait_gather(m, b):
        pltpu.make_async_copy(
            table_hbm.at[idx_v.at[pl.ds(m * _C, _C)]], rows_v.at[b], gsem[b]
        ).wait()

    def start_wb(m, b):
        pltpu.async_copy(
            rows_v.at[b], out_hbm.at[pl.ds(base + m * _C, _C)], wsem[b]
        )

    def wait_wb(m, b):
        pltpu.make_async_copy(
            rows_v.at[b], out_hbm.at[pl.ds(base + m * _C, _C)], wsem[b]
        ).wait()

    for j in range(_NBUF - 1):
        start_gather(j, j)

    def outer(g, carry):
        for b_s in range(_NBUF):
            m = g * _NBUF + b_s
            bn = (b_s + _NBUF - 1) % _NBUF

            @pl.when(m + (_NBUF - 1) < _STEPS)
            def _():
                @pl.when(m >= 1)
                def _():
                    wait_wb(m - 1, bn)

                start_gather(m + (_NBUF - 1), bn)

            wait_gather(m, b_s)
            start_wb(m, b_s)
        return carry

    lax.fori_loop(0, _STEPS // _NBUF, outer, 0)

    for j in range(_NBUF):
        wait_wb(_STEPS - _NBUF + j, (_STEPS - _NBUF + j) % _NBUF)


def kernel(input_ids, table):
    ids_flat = input_ids.reshape(-1).astype(jnp.int32)
    table_pad = jnp.pad(table[:, None, :], ((0, 0), (0, 1), (0, 0))).reshape(
        table.shape[0], _DP
    )
    out = _gather_kernel(table_pad, ids_flat)
    return out[:, :_D].reshape(input_ids.shape + (table.shape[1],))


# speedup vs baseline: 1.5863x; 1.1490x over previous
import functools

import jax
import jax.numpy as jnp
from jax import lax
from jax.experimental import pallas as pl
from jax.experimental.pallas import tpu as pltpu
from jax.experimental.pallas import tpu_sc as plsc

_INFO = plsc.get_sparse_core_info()
_NC, _NS = _INFO.num_cores, _INFO.num_subcores
_NW = _NC * _NS

_B = 4096 * 200
_D = 64
_DP = 128
_BPW = _B // _NW
_C = 400
_STEPS = _BPW // _C
_NBUF = 2


@functools.partial(
    pl.kernel,
    out_type=jax.ShapeDtypeStruct((_B, _DP), jnp.float32),
    mesh=plsc.VectorSubcoreMesh(core_axis_name="c", subcore_axis_name="s"),
    scratch_types=[
        pltpu.VMEM((_BPW,), jnp.int32),
        pltpu.VMEM((_NBUF, _C, _DP), jnp.float32),
    ]
    + [pltpu.SemaphoreType.DMA] * (2 * _NBUF),
)
def _gather_kernel(table_hbm, idx_hbm, out_hbm, idx_v, rows_v, *sems):
    gsem = sems[:_NBUF]
    wsem = sems[_NBUF:]
    wid = lax.axis_index("s") * _NC + lax.axis_index("c")
    base = wid * _BPW

    pltpu.sync_copy(idx_hbm.at[pl.ds(base, _BPW)], idx_v)

    def start_gather(m, b):
        pltpu.async_copy(
            table_hbm.at[idx_v.at[pl.ds(m * _C, _C)]], rows_v.at[b], gsem[b]
        )

    def wait_gather(m, b):
        pltpu.make_async_copy(
            table_hbm.at[idx_v.at[pl.ds(m * _C, _C)]], rows_v.at[b], gsem[b]
        ).wait()

    def start_wb(m, b):
        pltpu.async_copy(
            rows_v.at[b], out_hbm.at[pl.ds(base + m * _C, _C)], wsem[b]
        )

    def wait_wb(m, b):
        pltpu.make_async_copy(
            rows_v.at[b], out_hbm.at[pl.ds(base + m * _C, _C)], wsem[b]
        ).wait()

    for j in range(_NBUF - 1):
        start_gather(j, j)

    def outer(g, carry):
        for b_s in range(_NBUF):
            m = g * _NBUF + b_s
            bn = (b_s + _NBUF - 1) % _NBUF

            @pl.when(m + (_NBUF - 1) < _STEPS)
            def _():
                @pl.when(m >= 1)
                def _():
                    wait_wb(m - 1, bn)

                start_gather(m + (_NBUF - 1), bn)

            wait_gather(m, b_s)
            start_wb(m, b_s)
        return carry

    lax.fori_loop(0, _STEPS // _NBUF, outer, 0)

    for j in range(_NBUF):
        wait_wb(_STEPS - _NBUF + j, (_STEPS - _NBUF + j) % _NBUF)


def kernel(input_ids, table):
    ids_flat = input_ids.reshape(-1).astype(jnp.int32)
    table_pad = jnp.pad(table, ((0, 0), (0, _DP - _D)))
    out = _gather_kernel(table_pad, ids_flat)
    return out[:, :_D].reshape(input_ids.shape + (table.shape[1],))
